# baseline (speedup 1.0000x reference)
import functools

import jax
import jax.numpy as jnp
from jax.experimental import pallas as pl
from jax.experimental.pallas import tpu as pltpu

_B, _N, _C = 8, 262144, 2
_M = _B * _N
_LANES = 128
_ROWS = _M // _LANES
_BLK = 128


def _tc_body(lab_ref, lg_ref, w_ref, out_ref, acc_ref):
    i = pl.program_id(0)

    @pl.when(i == 0)
    def _():
        acc_ref[...] = jnp.zeros_like(acc_ref)

    lg = lg_ref[...]
    lab = lab_ref[...]
    w = w_ref[...]

    par = jax.lax.broadcasted_iota(jnp.int32, lg.shape, 1) & 1
    even = par == 0
    partner = jnp.where(even, jnp.roll(lg, -1, 1), jnp.roll(lg, 1, 1))
    m = jnp.maximum(lg, partner)
    lse = m + jnp.log(jnp.exp(lg - m) + jnp.exp(partner - m))

    labexp = jnp.repeat(lab, 2, axis=1)
    wexp = jnp.repeat(w, 2, axis=1)
    wm = jnp.where(labexp != -1, wexp, 0.0)
    pick = labexp == par
    g = wm * (0.5 * lse - jnp.where(pick, lg, 0.0))
    acc_ref[...] += jnp.sum(g.reshape(-1, 8, 2 * _LANES), axis=0)

    @pl.when(i == pl.num_programs(0) - 1)
    def _():
        out_ref[0, 0] = jnp.sum(acc_ref[...])


def kernel(rpn_labels, rpn_class_logits, rpn_label_weights):
    lab = rpn_labels.reshape(_ROWS, _LANES)
    lg = rpn_class_logits.reshape(_ROWS, 2 * _LANES)
    w = rpn_label_weights.reshape(_ROWS, _LANES)

    grid = (_ROWS // _BLK,)
    out = pl.pallas_call(
        _tc_body,
        grid=grid,
        in_specs=[
            pl.BlockSpec((_BLK, _LANES), lambda i: (i, 0)),
            pl.BlockSpec((_BLK, 2 * _LANES), lambda i: (i, 0)),
            pl.BlockSpec((_BLK, _LANES), lambda i: (i, 0)),
        ],
        out_specs=pl.BlockSpec(memory_space=pltpu.SMEM),
        out_shape=jax.ShapeDtypeStruct((1, 1), jnp.float32),
        scratch_shapes=[pltpu.VMEM((8, 2 * _LANES), jnp.float32)],
    )(lab, lg, w)
    return out[0, 0]


# speedup vs baseline: 1.4227x; 1.4227x over previous
import jax
import jax.numpy as jnp
import numpy as np
from jax.experimental import pallas as pl
from jax.experimental.pallas import tpu as pltpu

_B, _N, _C = 8, 262144, 2
_M = _B * _N
_LANES = 128
_ROWS = _M // _LANES
_BLK = 256

_EXPAND = np.zeros((128, 256), np.float32)
_EXPAND[np.arange(128), 2 * np.arange(128)] = 1.0
_EXPAND[np.arange(128), 2 * np.arange(128) + 1] = 1.0


def _tc_body(lab_ref, lg_ref, w_ref, e_ref, out_ref, acc_ref):
    i = pl.program_id(0)

    @pl.when(i == 0)
    def _():
        acc_ref[...] = jnp.zeros_like(acc_ref)

    lg = lg_ref[...]
    lab = lab_ref[...]
    w = w_ref[...]
    e = e_ref[...]

    par = jax.lax.broadcasted_iota(jnp.int32, lg.shape, 1) & 1
    even = par == 0
    dd = lg - jnp.roll(lg, -1, 1)
    du = lg - jnp.roll(lg, 1, 1)
    s = jnp.where(even, dd, du)
    d_abs = jnp.abs(s)
    phi = 0.5 * d_abs + jnp.log1p(jnp.exp(-d_abs))
    delta = jnp.where(even, -s, s)

    labf = lab.astype(jnp.float32)
    wm = jnp.where(lab != -1, w, 0.0)
    c = wm * (labf - 0.5)

    dims = (((1,), (0,)), ((), ()))
    wmexp = jax.lax.dot_general(wm.astype(jnp.bfloat16), e, dims,
                                preferred_element_type=jnp.float32)
    cexp = jax.lax.dot_general(c.astype(jnp.bfloat16), e, dims,
                               preferred_element_type=jnp.float32)

    g = wmexp * phi - cexp * delta
    acc_ref[...] += jnp.sum(g.reshape(-1, 8, 2 * _LANES), axis=0)

    @pl.when(i == pl.num_programs(0) - 1)
    def _():
        out_ref[0, 0] = 0.5 * jnp.sum(acc_ref[...])


def kernel(rpn_labels, rpn_class_logits, rpn_label_weights):
    lab = rpn_labels.reshape(_ROWS, _LANES)
    lg = rpn_class_logits.reshape(_ROWS, 2 * _LANES)
    w = rpn_label_weights.reshape(_ROWS, _LANES)
    e = jnp.asarray(_EXPAND, dtype=jnp.bfloat16)

    grid = (_ROWS // _BLK,)
    out = pl.pallas_call(
        _tc_body,
        grid=grid,
        in_specs=[
            pl.BlockSpec((_BLK, _LANES), lambda i: (i, 0)),
            pl.BlockSpec((_BLK, 2 * _LANES), lambda i: (i, 0)),
            pl.BlockSpec((_BLK, _LANES), lambda i: (i, 0)),
            pl.BlockSpec((_LANES, 2 * _LANES), lambda i: (0, 0)),
        ],
        out_specs=pl.BlockSpec(memory_space=pltpu.SMEM),
        out_shape=jax.ShapeDtypeStruct((1, 1), jnp.float32),
        scratch_shapes=[pltpu.VMEM((8, 2 * _LANES), jnp.float32)],
    )(lab, lg, w, e)
    return out[0, 0]


# speedup vs baseline: 139.1031x; 97.7722x over previous
import jax
import jax.numpy as jnp
from jax.experimental import pallas as pl
from jax.experimental.pallas import tpu as pltpu

_B, _N = 8, 262144
_LANES = 128
_NH = _N // _LANES
_CH = 64


def _tc_body(lab_ref, lg_ref, w_ref, out_ref, acc_ref):
    i = pl.program_id(0)

    @pl.when(i == 0)
    def _():
        acc_ref[...] = jnp.zeros_like(acc_ref)

    lab = lab_ref[...].reshape(_B, _CH, _LANES)
    w = w_ref[...].reshape(_B, _CH, _LANES)
    l0 = lg_ref[:, 0::2, :]
    l1 = lg_ref[:, 1::2, :]

    m = jnp.maximum(l0, l1)
    mn = jnp.minimum(l0, l1)
    lse = m + jnp.log1p(jnp.exp(mn - m))
    sel = jnp.where(lab == 1, l1, l0)
    wm = jnp.where(lab != -1, w, 0.0)
    g = (lse - sel) * wm
    acc_ref[...] += jnp.sum(g, axis=1)

    @pl.when(i == pl.num_programs(0) - 1)
    def _():
        out_ref[0, 0] = jnp.sum(acc_ref[...])


def kernel(rpn_labels, rpn_class_logits, rpn_label_weights):
    lg = rpn_class_logits.reshape(_B, _NH, _LANES, 2)
    lg = lg.transpose(0, 1, 3, 2).reshape(_B, 2 * _NH, _LANES)

    grid = (_NH // _CH,)
    out = pl.pallas_call(
        _tc_body,
        grid=grid,
        in_specs=[
            pl.BlockSpec((_B, _CH * _LANES), lambda i: (0, i)),
            pl.BlockSpec((_B, 2 * _CH, _LANES), lambda i: (0, i, 0)),
            pl.BlockSpec((_B, _CH * _LANES), lambda i: (0, i)),
        ],
        out_specs=pl.BlockSpec(memory_space=pltpu.SMEM),
        out_shape=jax.ShapeDtypeStruct((1, 1), jnp.float32),
        scratch_shapes=[pltpu.VMEM((_B, _LANES), jnp.float32)],
    )(rpn_labels, lg, rpn_label_weights)
    return out[0, 0]


# speedup vs baseline: 144.1690x; 1.0364x over previous
import jax
import jax.numpy as jnp
from jax.experimental import pallas as pl
from jax.experimental.pallas import tpu as pltpu

_B, _N = 8, 262144
_LANES = 128
_NH = _N // _LANES
_CH = 64


def _tc_body(lab_ref, lg_ref, w_ref, out_ref, acc_ref):
    i = pl.program_id(0)

    @pl.when(i == 0)
    def _():
        acc_ref[...] = jnp.zeros_like(acc_ref)

    for s in range(_CH // 8):
        l0 = lg_ref[:, pl.Slice(16 * s, 8, 2), :]
        l1 = lg_ref[:, pl.Slice(16 * s + 1, 8, 2), :]
        lab = lab_ref[:, pl.ds(1024 * s, 1024)].reshape(_B, 8, _LANES)
        w = w_ref[:, pl.ds(1024 * s, 1024)].reshape(_B, 8, _LANES)
        d = l1 - l0
        sp = jnp.log1p(jnp.exp(-jnp.abs(d)))
        ce = jnp.maximum(d, 0.0) - lab.astype(jnp.float32) * d + sp
        wm = jnp.where(lab != -1, w, 0.0)
        acc_ref[...] += jnp.sum(ce * wm, axis=1)

    @pl.when(i == pl.num_programs(0) - 1)
    def _():
        out_ref[0, 0] = jnp.sum(acc_ref[...])


def kernel(rpn_labels, rpn_class_logits, rpn_label_weights):
    lg = rpn_class_logits.reshape(_B, _NH, _LANES, 2)
    lg = lg.transpose(0, 1, 3, 2).reshape(_B, 2 * _NH, _LANES)

    grid = (_NH // _CH,)
    out = pl.pallas_call(
        _tc_body,
        grid=grid,
        in_specs=[
            pl.BlockSpec((_B, _CH * _LANES), lambda i: (0, i)),
            pl.BlockSpec((_B, 2 * _CH, _LANES), lambda i: (0, i, 0)),
            pl.BlockSpec((_B, _CH * _LANES), lambda i: (0, i)),
        ],
        out_specs=pl.BlockSpec(memory_space=pltpu.SMEM),
        out_shape=jax.ShapeDtypeStruct((1, 1), jnp.float32),
        scratch_shapes=[pltpu.VMEM((_B, _LANES), jnp.float32)],
    )(rpn_labels, lg, rpn_label_weights)
    return out[0, 0]
